# baseline (speedup 1.0000x reference)
import jax
import jax.numpy as jnp
from jax import lax
from jax.experimental import pallas as pl
from jax.experimental.pallas import tpu as pltpu
from jax.experimental.pallas import tpu_sc as plsc

B, N, K, E = 4, 2500, 12, 7500
NW = 32
WPB = NW // B
F = N * K
FP = 30720
CH = FP // WPB
NPAD = FP // K
EP = 7680
GROUPS = CH // 16
UNROLL = 8


def _sc_gather(oe_hbm, ve_hbm, ef_hbm, oe_v, ve_v, ef_v, sem):
    c = lax.axis_index("c")
    s = lax.axis_index("s")
    wid = s * 2 + c
    batch = wid // WPB
    sub = lax.rem(wid, WPB)

    cp0 = pltpu.async_copy(oe_hbm.at[batch], oe_v, sem)
    cp1 = pltpu.async_copy(ve_hbm.at[batch, sub], ve_v, sem)
    cp0.wait()
    cp1.wait()

    def group(g, carry):
        for u in range(UNROLL):
            p = (g * UNROLL + u) * 16
            ef_v[pl.ds(p, 16)] = plsc.load_gather(oe_v, [ve_v[pl.ds(p, 16)]])
        return carry

    lax.fori_loop(0, GROUPS // UNROLL, group, 0)
    pltpu.sync_copy(ef_v, ef_hbm.at[batch, sub])


def _tc_loss(ef_ref, vs_ref, gt_ref, out_ref):
    off = jnp.mean(ef_ref[...], axis=-1)
    d = vs_ref[...] - gt_ref[...]
    pv = jnp.mean((d + off[:, :N, None]) ** 2, axis=-1)
    out_ref[0, 0] = jnp.sum(pv)


@jax.jit
def kernel(out_edges, gt_vs, vs, ve):
    oe = jnp.pad(out_edges[:, 0, :], ((0, 0), (0, EP - E)))
    ve_p = jnp.pad(ve.reshape(B, F), ((0, 0), (0, FP - F)))
    ve_r = ve_p.reshape(B, WPB, CH)

    mesh = plsc.VectorSubcoreMesh(core_axis_name="c", subcore_axis_name="s")
    gather = pl.kernel(
        _sc_gather,
        out_type=jax.ShapeDtypeStruct((B, WPB, CH), jnp.float32),
        mesh=mesh,
        compiler_params=pltpu.CompilerParams(needs_layout_passes=False),
        scratch_types=[
            pltpu.VMEM((EP,), jnp.float32),
            pltpu.VMEM((CH,), jnp.int32),
            pltpu.VMEM((CH,), jnp.float32),
            pltpu.SemaphoreType.DMA,
        ],
    )
    e_f = gather(oe, ve_r)

    loss = pl.pallas_call(
        _tc_loss,
        out_shape=jax.ShapeDtypeStruct((1, 1), jnp.float32),
        in_specs=[
            pl.BlockSpec(memory_space=pltpu.VMEM),
            pl.BlockSpec(memory_space=pltpu.VMEM),
            pl.BlockSpec(memory_space=pltpu.VMEM),
        ],
        out_specs=pl.BlockSpec(memory_space=pltpu.SMEM),
    )(e_f.reshape(B, NPAD, K), vs, gt_vs)
    return loss[0, 0]


# speedup vs baseline: 1.8592x; 1.8592x over previous
import jax
import jax.numpy as jnp
from jax import lax
from jax.experimental import pallas as pl
from jax.experimental.pallas import tpu as pltpu
from jax.experimental.pallas import tpu_sc as plsc

B, N, K, E = 4, 2500, 12, 7500
NW = 32
WPB = NW // B
NP = 2560
CH = NP // WPB
EP = 7680
GROUPS = CH // 16


def _sc_loss(oe_hbm, ve_hbm, vg_hbm, out_hbm,
             oe_v, ve_v, vg_v, loss_v, sem):
    c = lax.axis_index("c")
    s = lax.axis_index("s")
    wid = s * 2 + c
    batch = wid // WPB
    sub = lax.rem(wid, WPB)

    cp0 = pltpu.async_copy(oe_hbm.at[batch], oe_v, sem)
    cp1 = pltpu.async_copy(ve_hbm.at[batch, sub], ve_v, sem)
    cp2 = pltpu.async_copy(vg_hbm.at[batch, sub], vg_v, sem)
    cp0.wait()
    cp1.wait()
    cp2.wait()

    def group(g, acc):
        jb = g * 16
        osum = plsc.load_gather(oe_v, [ve_v[0, pl.ds(jb, 16)]])
        for k in range(1, K):
            osum = osum + plsc.load_gather(oe_v, [ve_v[k, pl.ds(jb, 16)]])
        off = osum * (1.0 / K)
        d0 = vg_v[0, pl.ds(jb, 16)] - vg_v[3, pl.ds(jb, 16)]
        d1 = vg_v[1, pl.ds(jb, 16)] - vg_v[4, pl.ds(jb, 16)]
        d2 = vg_v[2, pl.ds(jb, 16)] - vg_v[5, pl.ds(jb, 16)]
        a = (d0 * d0 + d1 * d1 + d2 * d2) * (1.0 / 3.0)
        bd = (d0 + d1 + d2) * (2.0 / 3.0)
        return acc + a + off * (bd + off)

    loss16 = lax.fori_loop(0, GROUPS, group, jnp.zeros((16,), jnp.float32))
    loss_v[...] = loss16
    pltpu.sync_copy(loss_v, out_hbm.at[wid])


@jax.jit
def kernel(out_edges, gt_vs, vs, ve):
    oe = jnp.pad(out_edges[:, 0, :], ((0, 0), (0, EP - E)))
    ve_t = jnp.pad(ve.transpose(0, 2, 1), ((0, 0), (0, 0), (0, NP - N)),
                   constant_values=E)
    ve_r = ve_t.reshape(B, K, WPB, CH).transpose(0, 2, 1, 3)
    vg = jnp.concatenate([vs, gt_vs], axis=2)
    vg_t = jnp.pad(vg.transpose(0, 2, 1), ((0, 0), (0, 0), (0, NP - N)))
    vg_r = vg_t.reshape(B, 6, WPB, CH).transpose(0, 2, 1, 3)

    mesh = plsc.VectorSubcoreMesh(core_axis_name="c", subcore_axis_name="s")
    run = pl.kernel(
        _sc_loss,
        out_type=jax.ShapeDtypeStruct((NW, 16), jnp.float32),
        mesh=mesh,
        compiler_params=pltpu.CompilerParams(needs_layout_passes=False),
        scratch_types=[
            pltpu.VMEM((EP,), jnp.float32),
            pltpu.VMEM((K, CH), jnp.int32),
            pltpu.VMEM((6, CH), jnp.float32),
            pltpu.VMEM((16,), jnp.float32),
            pltpu.SemaphoreType.DMA,
        ],
    )
    partials = run(oe, ve_r, vg_r)
    return jnp.sum(partials)


# speedup vs baseline: 1.9243x; 1.0350x over previous
import jax
import jax.numpy as jnp
from jax import lax
from jax.experimental import pallas as pl
from jax.experimental.pallas import tpu as pltpu
from jax.experimental.pallas import tpu_sc as plsc

B, N, K, E = 4, 2500, 12, 7500
NW = 32
WPB = NW // B
NP = 2560
CH = NP // WPB
GROUPS = CH // 16


def _sc_loss(oe_hbm, ve_hbm, vg_hbm, out_hbm,
             oe_v, ve_v, vg_v, loss_v, sem):
    c = lax.axis_index("c")
    s = lax.axis_index("s")
    wid = s * 2 + c
    batch = wid // WPB
    sub = lax.rem(wid, WPB)

    cp0 = pltpu.async_copy(oe_hbm.at[batch, 0], oe_v, sem)
    cp1 = pltpu.async_copy(ve_hbm.at[batch, sub], ve_v, sem)
    cp2 = pltpu.async_copy(vg_hbm.at[batch, sub], vg_v, sem)
    cp0.wait()
    cp1.wait()
    cp2.wait()

    gid0 = sub * CH + lax.iota(jnp.int32, 16)

    def group(g, acc):
        jb = g * 16
        osum = plsc.load_gather(oe_v, [ve_v[0, pl.ds(jb, 16)]])
        for k in range(1, K):
            osum = osum + plsc.load_gather(oe_v, [ve_v[k, pl.ds(jb, 16)]])
        off = osum * (1.0 / K)
        d0 = vg_v[0, pl.ds(jb, 16)] - vg_v[3, pl.ds(jb, 16)]
        d1 = vg_v[1, pl.ds(jb, 16)] - vg_v[4, pl.ds(jb, 16)]
        d2 = vg_v[2, pl.ds(jb, 16)] - vg_v[5, pl.ds(jb, 16)]
        a = (d0 * d0 + d1 * d1 + d2 * d2) * (1.0 / 3.0)
        bd = (d0 + d1 + d2) * (2.0 / 3.0)
        contrib = a + off * (bd + off)
        contrib = jnp.where(gid0 + jb < N, contrib, 0.0)
        return acc + contrib

    loss16 = lax.fori_loop(0, GROUPS, group, jnp.zeros((16,), jnp.float32))
    loss_v[...] = loss16
    pltpu.sync_copy(loss_v, out_hbm.at[wid])


@jax.jit
def kernel(out_edges, gt_vs, vs, ve):
    ve_t = jnp.pad(ve.transpose(0, 2, 1), ((0, 0), (0, 0), (0, NP - N)))
    ve_r = ve_t.reshape(B, K, WPB, CH).transpose(0, 2, 1, 3)
    vg = jnp.concatenate([vs, gt_vs], axis=2)
    vg_t = jnp.pad(vg.transpose(0, 2, 1), ((0, 0), (0, 0), (0, NP - N)))
    vg_r = vg_t.reshape(B, 6, WPB, CH).transpose(0, 2, 1, 3)

    mesh = plsc.VectorSubcoreMesh(core_axis_name="c", subcore_axis_name="s")
    run = pl.kernel(
        _sc_loss,
        out_type=jax.ShapeDtypeStruct((NW, 16), jnp.float32),
        mesh=mesh,
        compiler_params=pltpu.CompilerParams(needs_layout_passes=False),
        scratch_types=[
            pltpu.VMEM((E,), jnp.float32),
            pltpu.VMEM((K, CH), jnp.int32),
            pltpu.VMEM((6, CH), jnp.float32),
            pltpu.VMEM((16,), jnp.float32),
            pltpu.SemaphoreType.DMA,
        ],
    )
    partials = run(out_edges, ve_r, vg_r)
    return jnp.sum(partials)
